# baseline (speedup 1.0000x reference)
import functools

import jax
import jax.numpy as jnp
from jax import lax
from jax.experimental import pallas as pl
from jax.experimental.pallas import tpu as pltpu
from jax.experimental.pallas import tpu_sc as plsc

BATCH = 16384
DIM = 64
NC, NS = 2, 16
NW = NC * NS
B_PER_W = BATCH // NW
CHUNK = 128
N_CHUNKS = B_PER_W // CHUNK
QROWS = 250000
QW = 128


def _pack_quad(table):
    b = lax.bitcast_convert_type(table.astype(jnp.bfloat16), jnp.uint16)
    b = b.astype(jnp.uint32).reshape(QROWS, 4, 32, 2)
    packed = b[..., 0] | (b[..., 1] << 16)
    return packed.reshape(QROWS, QW)


def _sc_gather(uq2, iq2, gu_q, gi_q, mu_q, mi_q):
    mesh = plsc.VectorSubcoreMesh(core_axis_name="c", subcore_axis_name="s")

    @functools.partial(
        pl.kernel,
        out_type=[jax.ShapeDtypeStruct((BATCH, QW), jnp.uint32)] * 4,
        mesh=mesh,
        scratch_types=[
            pltpu.VMEM((N_CHUNKS, CHUNK), jnp.int32),
            pltpu.VMEM((N_CHUNKS, CHUNK), jnp.int32),
            pltpu.VMEM((CHUNK, QW), jnp.uint32),
            pltpu.VMEM((CHUNK, QW), jnp.uint32),
            pltpu.SemaphoreType.DMA,
            pltpu.SemaphoreType.DMA,
        ],
    )
    def k(uq_hbm, iq_hbm, gu_hbm, gi_hbm, mu_hbm, mi_hbm,
          gu_out, gi_out, mu_out, mi_out,
          uq_v, iq_v, buf_a, buf_b, sem_a, sem_b):
        wid = lax.axis_index("s") * NC + lax.axis_index("c")
        crow = wid * N_CHUNKS
        base = wid * B_PER_W
        pltpu.sync_copy(uq_hbm.at[pl.ds(crow, N_CHUNKS)], uq_v)
        pltpu.sync_copy(iq_hbm.at[pl.ds(crow, N_CHUNKS)], iq_v)

        jobs = []
        for table, idx_v, out in ((gu_hbm, uq_v, gu_out),
                                  (gi_hbm, iq_v, gi_out),
                                  (mu_hbm, uq_v, mu_out),
                                  (mi_hbm, iq_v, mi_out)):
            for j in range(N_CHUNKS):
                jobs.append((table, idx_v, out, j))

        bufs = (buf_a, buf_b)
        sems = (sem_a, sem_b)
        prev = None
        for n, (table, idx_v, out, j) in enumerate(jobs):
            s = n % 2
            cp = pltpu.async_copy(table.at[idx_v.at[j]], bufs[s], sems[s])
            if prev is not None:
                p_s, p_out, p_off, p_cp = prev
                p_cp.wait()
                pltpu.sync_copy(bufs[p_s], p_out.at[pl.ds(p_off, CHUNK)])
            prev = (s, out, base + j * CHUNK, cp)
        p_s, p_out, p_off, p_cp = prev
        p_cp.wait()
        pltpu.sync_copy(bufs[p_s], p_out.at[pl.ds(p_off, CHUNK)])

    return k(uq2, iq2, gu_q, gi_q, mu_q, mi_q)


BM = 2048


def _select_unpack(slab_ref, sel):
    x = slab_ref[...]
    g = jnp.zeros((BM, 32), jnp.uint32)
    for k in range(4):
        g = g | jnp.where(sel[:, k:k + 1] != 0, x[:, 32 * k:32 * (k + 1)], 0)
    even = lax.bitcast_convert_type(g << 16, jnp.float32)
    odd = lax.bitcast_convert_type(g & jnp.uint32(0xFFFF0000), jnp.float32)
    return even, odd


def _tc_mlp(gu_s, gi_s, mu_s, mi_s, selu, seli,
            W1e_u, W1o_u, W1e_i, W1o_i, b1, W2, b2, wfe, wfo, wf_h, bf):
    def body(gu_ref, gi_ref, mu_ref, mi_ref, selu_ref, seli_ref,
             w1eu_ref, w1ou_ref, w1ei_ref, w1oi_ref, b1_ref,
             w2_ref, b2_ref, wfe_ref, wfo_ref, wfh_ref, bf_ref, out_ref):
        su = selu_ref[...]
        si = seli_ref[...]
        gue, guo = _select_unpack(gu_ref, su)
        gie, gio = _select_unpack(gi_ref, si)
        mue, muo = _select_unpack(mu_ref, su)
        mie, mio = _select_unpack(mi_ref, si)
        f32 = jnp.float32
        h = (jnp.dot(mue, w1eu_ref[...], preferred_element_type=f32)
             + jnp.dot(muo, w1ou_ref[...], preferred_element_type=f32)
             + jnp.dot(mie, w1ei_ref[...], preferred_element_type=f32)
             + jnp.dot(mio, w1oi_ref[...], preferred_element_type=f32))
        h = jnp.maximum(h + b1_ref[...], 0.0)
        h = jnp.maximum(
            jnp.dot(h, w2_ref[...], preferred_element_type=f32)
            + b2_ref[...], 0.0)
        pred = (jnp.dot(gue * gie, wfe_ref[...], preferred_element_type=f32)
                + jnp.dot(guo * gio, wfo_ref[...], preferred_element_type=f32)
                + jnp.dot(h, wfh_ref[...], preferred_element_type=f32)
                + bf_ref[...])
        out_ref[...] = pred

    grid = (BATCH // BM,)
    slab_spec = pl.BlockSpec((BM, QW), lambda i: (i, 0))
    sel_spec = pl.BlockSpec((BM, 4), lambda i: (i, 0))
    full = lambda shape: pl.BlockSpec(shape, lambda i: (0,) * len(shape))
    return pl.pallas_call(
        body,
        grid=grid,
        in_specs=[
            slab_spec, slab_spec, slab_spec, slab_spec,
            sel_spec, sel_spec,
            full((32, DIM)), full((32, DIM)), full((32, DIM)), full((32, DIM)),
            full((1, DIM)),
            full((DIM, 32)), full((1, 32)),
            full((32, 1)), full((32, 1)), full((32, 1)), full((1, 1)),
        ],
        out_specs=pl.BlockSpec((BM, 1), lambda i: (i, 0)),
        out_shape=jax.ShapeDtypeStruct((BATCH, 1), jnp.float32),
    )(gu_s, gi_s, mu_s, mi_s, selu, seli,
      W1e_u, W1o_u, W1e_i, W1o_i, b1, W2, b2, wfe, wfo, wf_h, bf)


def kernel(user_ids, item_ids, gmf_user_w, gmf_item_w, mlp_user_w, mlp_item_w,
           W1, b1, W2, b2, Wf, bf):
    uidx = user_ids.astype(jnp.int32)
    iidx = item_ids.astype(jnp.int32)
    uq2 = (uidx // 4).reshape(BATCH // CHUNK, CHUNK)
    iq2 = (iidx // 4).reshape(BATCH // CHUNK, CHUNK)
    selu = (jnp.arange(4, dtype=jnp.int32)[None, :]
            == (uidx % 4)[:, None]).astype(jnp.int32)
    seli = (jnp.arange(4, dtype=jnp.int32)[None, :]
            == (iidx % 4)[:, None]).astype(jnp.int32)

    gu_s, gi_s, mu_s, mi_s = _sc_gather(
        uq2, iq2, _pack_quad(gmf_user_w), _pack_quad(gmf_item_w),
        _pack_quad(mlp_user_w), _pack_quad(mlp_item_w))

    pred = _tc_mlp(
        gu_s, gi_s, mu_s, mi_s, selu, seli,
        W1[0:DIM:2], W1[1:DIM:2], W1[DIM::2], W1[DIM + 1::2],
        b1.reshape(1, DIM), W2, b2.reshape(1, 32),
        Wf[0:DIM:2], Wf[1:DIM:2], Wf[DIM:], bf.reshape(1, 1))
    return pred[:, 0]


# speedup vs baseline: 35.3414x; 35.3414x over previous
import functools

import jax
import jax.numpy as jnp
from jax import lax
from jax.experimental import pallas as pl
from jax.experimental.pallas import tpu as pltpu
from jax.experimental.pallas import tpu_sc as plsc

BATCH = 16384
DIM = 64
NUSERS = 1000000
NC, NS = 2, 16
NW = NC * NS
B_PER_W = BATCH // NW
CHUNK = 128
N_CHUNKS = B_PER_W // CHUNK
PROWS = NUSERS // 2


def _sc_gather(up2, ip2, gu_p, gi_p, mu_p, mi_p):
    mesh = plsc.VectorSubcoreMesh(core_axis_name="c", subcore_axis_name="s")

    @functools.partial(
        pl.kernel,
        out_type=[jax.ShapeDtypeStruct((BATCH, 128), jnp.float32)] * 4,
        mesh=mesh,
        scratch_types=[
            pltpu.VMEM((N_CHUNKS, CHUNK), jnp.int32),
            pltpu.VMEM((N_CHUNKS, CHUNK), jnp.int32),
            pltpu.VMEM((CHUNK, 128), jnp.float32),
            pltpu.VMEM((CHUNK, 128), jnp.float32),
            pltpu.SemaphoreType.DMA,
            pltpu.SemaphoreType.DMA,
        ],
    )
    def k(up_hbm, ip_hbm, gu_hbm, gi_hbm, mu_hbm, mi_hbm,
          gu_out, gi_out, mu_out, mi_out,
          up_v, ip_v, buf_a, buf_b, sem_a, sem_b):
        wid = lax.axis_index("s") * NC + lax.axis_index("c")
        crow = wid * N_CHUNKS
        base = wid * B_PER_W
        pltpu.sync_copy(up_hbm.at[pl.ds(crow, N_CHUNKS)], up_v)
        pltpu.sync_copy(ip_hbm.at[pl.ds(crow, N_CHUNKS)], ip_v)

        jobs = []
        for table, idx_v, out in ((gu_hbm, up_v, gu_out),
                                  (gi_hbm, ip_v, gi_out),
                                  (mu_hbm, up_v, mu_out),
                                  (mi_hbm, ip_v, mi_out)):
            for j in range(N_CHUNKS):
                jobs.append((table, idx_v, out, j))

        bufs = (buf_a, buf_b)
        sems = (sem_a, sem_b)
        prev = None
        for n, (table, idx_v, out, j) in enumerate(jobs):
            s = n % 2
            cp = pltpu.async_copy(table.at[idx_v.at[j]], bufs[s], sems[s])
            if prev is not None:
                p_s, p_out, p_off, p_cp = prev
                p_cp.wait()
                pltpu.sync_copy(bufs[p_s], p_out.at[pl.ds(p_off, CHUNK)])
            prev = (s, out, base + j * CHUNK, cp)
        p_s, p_out, p_off, p_cp = prev
        p_cp.wait()
        pltpu.sync_copy(bufs[p_s], p_out.at[pl.ds(p_off, CHUNK)])

    return k(up2, ip2, gu_p, gi_p, mu_p, mi_p)


BM = 2048


def _sel_half(slab_ref, sel2):
    x = slab_ref[...]
    m0 = (sel2[:, 0:1] != 0).astype(jnp.float32)
    m1 = (sel2[:, 1:2] != 0).astype(jnp.float32)
    return x[:, :DIM] * m0 + x[:, DIM:] * m1


def _tc_mlp(gu_s, gi_s, mu_s, mi_s, selu2, seli2, W1, b1, W2, b2, Wf, bf):
    def body(gu_ref, gi_ref, mu_ref, mi_ref, selu_ref, seli_ref,
             w1_ref, b1_ref, w2_ref, b2_ref, wf_ref, bf_ref, out_ref):
        su = selu_ref[...]
        si = seli_ref[...]
        gmf = _sel_half(gu_ref, su) * _sel_half(gi_ref, si)
        mu = _sel_half(mu_ref, su)
        mi = _sel_half(mi_ref, si)
        f32 = jnp.float32
        w1 = w1_ref[...]
        h = (jnp.dot(mu, w1[:DIM], preferred_element_type=f32)
             + jnp.dot(mi, w1[DIM:], preferred_element_type=f32))
        h = jnp.maximum(h + b1_ref[...], 0.0)
        h = jnp.maximum(
            jnp.dot(h, w2_ref[...], preferred_element_type=f32)
            + b2_ref[...], 0.0)
        wf = wf_ref[...]
        pred = (jnp.dot(gmf, wf[:DIM], preferred_element_type=f32)
                + jnp.dot(h, wf[DIM:], preferred_element_type=f32)
                + bf_ref[...])
        out_ref[...] = pred

    grid = (BATCH // BM,)
    slab_spec = pl.BlockSpec((BM, 128), lambda i: (i, 0))
    sel_spec = pl.BlockSpec((BM, 2), lambda i: (i, 0))
    full = lambda shape: pl.BlockSpec(shape, lambda i: (0,) * len(shape))
    return pl.pallas_call(
        body,
        grid=grid,
        in_specs=[
            slab_spec, slab_spec, slab_spec, slab_spec,
            sel_spec, sel_spec,
            full((2 * DIM, DIM)), full((1, DIM)),
            full((DIM, 32)), full((1, 32)),
            full((DIM + 32, 1)), full((1, 1)),
        ],
        out_specs=pl.BlockSpec((BM, 1), lambda i: (i, 0)),
        out_shape=jax.ShapeDtypeStruct((BATCH, 1), jnp.float32),
    )(gu_s, gi_s, mu_s, mi_s, selu2, seli2, W1, b1, W2, b2, Wf, bf)


def _onehot(v, n):
    return (jnp.arange(n, dtype=jnp.int32)[None, :]
            == v[:, None]).astype(jnp.int32)


def kernel(user_ids, item_ids, gmf_user_w, gmf_item_w, mlp_user_w, mlp_item_w,
           W1, b1, W2, b2, Wf, bf):
    uidx = user_ids.astype(jnp.int32)
    iidx = item_ids.astype(jnp.int32)
    shp = (BATCH // CHUNK, CHUNK)
    up2, ip2 = (uidx // 2).reshape(shp), (iidx // 2).reshape(shp)
    selu2, seli2 = _onehot(uidx % 2, 2), _onehot(iidx % 2, 2)

    gu_s, gi_s, mu_s, mi_s = _sc_gather(
        up2, ip2,
        gmf_user_w.reshape(PROWS, 128), gmf_item_w.reshape(PROWS, 128),
        mlp_user_w.reshape(PROWS, 128), mlp_item_w.reshape(PROWS, 128))

    pred = _tc_mlp(gu_s, gi_s, mu_s, mi_s, selu2, seli2,
                   W1, b1.reshape(1, DIM), W2, b2.reshape(1, 32),
                   Wf, bf.reshape(1, 1))
    return pred[:, 0]
